# Initial kernel scaffold; baseline (speedup 1.0000x reference)
import jax
import jax.numpy as jnp
from jax.experimental import pallas as pl


def kernel(coords, vector_field):
    raise NotImplementedError("write your pallas kernel here")


# speedup vs baseline: 30.6109x; 30.6109x over previous
import functools

import jax
import jax.numpy as jnp
from jax import lax
from jax.experimental import pallas as pl
from jax.experimental.pallas import tpu as pltpu
from jax.experimental.pallas import tpu_sc as plsc

H, W, C = 1024, 1024, 2

NC = 2
NS = 16
L = 16
NW = NC * NS

B = 2048
NSTR = B // 128
NG = B // L


def _sc_body(npoints, nblocks, coords_hbm, grid8_hbm, out_hbm,
             cbuf, ibuf, wxbuf, wybuf, gbuf, obuf, gsem):
    per_worker = npoints // NW
    ids = lax.iota(jnp.int32, L)
    wid = lax.axis_index("s") * NC + lax.axis_index("c")
    base_pt = wid * per_worker

    def block(b, carry):
        blk0 = base_pt + b * B
        pltpu.sync_copy(coords_hbm.at[pl.ds(blk0 * 2, B * 2)], cbuf)

        def p1(gi, _):
            xi = ids * 2 + gi * (2 * L)
            x = plsc.load_gather(cbuf, [xi])
            y = plsc.load_gather(cbuf, [xi + 1])
            xs = x * jnp.float32(W - 1)
            ys = y * jnp.float32(H - 1)
            x0 = xs.astype(jnp.int32)
            y0 = ys.astype(jnp.int32)
            wx = xs - x0.astype(jnp.float32)
            wy = ys - y0.astype(jnp.float32)
            r = y0 * W + x0
            ibuf[pl.ds(gi * L, L)] = r
            wxbuf[pl.ds(gi * L, L)] = wx
            wybuf[pl.ds(gi * L, L)] = wy
            return _

        lax.fori_loop(0, NG, p1, 0)

        copies = []
        for j in range(NSTR):
            copies.append(pltpu.async_copy(
                grid8_hbm.at[ibuf.at[pl.ds(j * 128, 128)]],
                gbuf.at[pl.ds(j * 128, 128)], gsem))
        for cp in copies:
            cp.wait()

        def p3(gi, _):
            bse = gi * L
            rows = ids + bse
            gv = [plsc.load_gather(gbuf, [rows, jnp.full((L,), k, jnp.int32)])
                  for k in range(8)]
            wx = wxbuf[pl.ds(bse, L)]
            wy = wybuf[pl.ds(bse, L)]
            top0 = gv[0] + wx * (gv[2] - gv[0])
            top1 = gv[1] + wx * (gv[3] - gv[1])
            bot0 = gv[4] + wx * (gv[6] - gv[4])
            bot1 = gv[5] + wx * (gv[7] - gv[5])
            o0 = top0 + wy * (bot0 - top0)
            o1 = top1 + wy * (bot1 - top1)
            oi = ids * 2 + bse * 2
            plsc.store_scatter(obuf, [oi], o0)
            plsc.store_scatter(obuf, [oi + 1], o1)
            return _

        lax.fori_loop(0, NG, p3, 0)
        pltpu.sync_copy(obuf, out_hbm.at[pl.ds(blk0 * 2, B * 2)])
        return carry

    lax.fori_loop(0, nblocks, block, 0)


def _sample(flat_coords, grid8, npoints):
    per_worker = npoints // NW
    nblocks = per_worker // B
    mesh = plsc.VectorSubcoreMesh(core_axis_name="c", subcore_axis_name="s")
    body = functools.partial(_sc_body, npoints, nblocks)
    return pl.kernel(
        body,
        out_type=jax.ShapeDtypeStruct((npoints * 2,), jnp.float32),
        mesh=mesh,
        compiler_params=pltpu.CompilerParams(
            needs_layout_passes=False, use_tc_tiling_on_sc=False),
        scratch_types=[
            pltpu.VMEM((B * 2,), jnp.float32),
            pltpu.VMEM((B,), jnp.int32),
            pltpu.VMEM((B,), jnp.float32),
            pltpu.VMEM((B,), jnp.float32),
            pltpu.VMEM((B, 8), jnp.float32),
            pltpu.VMEM((B * 2,), jnp.float32),
            pltpu.SemaphoreType.DMA,
        ],
    )(flat_coords, grid8)


def kernel(coords, vector_field):
    shape = coords.shape
    npoints = coords.size // shape[-1]
    flat_coords = coords.reshape(-1)
    g = vector_field
    gx = jnp.roll(g, -1, axis=1)
    gy = jnp.roll(g, -1, axis=0)
    gxy = jnp.roll(gy, -1, axis=1)
    grid8 = jnp.concatenate([g, gx, gy, gxy], axis=-1).reshape(H * W, 8)
    out_flat = _sample(flat_coords, grid8, npoints)
    return out_flat.reshape(*shape[:-1], C)
